# baseline (speedup 1.0000x reference)
import dataclasses
import functools

import jax
import jax.numpy as jnp
from jax import lax
from jax.experimental import pallas as pl
from jax.experimental.pallas import tpu as pltpu
from jax.experimental.pallas import tpu_sc as plsc

B = 16384
L = 50
D = 64
NCAT = 1000
ALPHA = 0.1

NW = 32
NSUB = 16
TPW = B * L // NW
K = 128
NCHUNK = TPW // K
SPW = B // NW
PASSES = 4
SPP = SPW // PASSES
CPP = NCHUNK // PASSES
NLANE = 16
DEPTH = 5


def _build(dh):
    mesh = plsc.VectorSubcoreMesh(core_axis_name="c", subcore_axis_name="s")
    cp = pltpu.CompilerParams(use_tc_tiling_on_sc=False)
    if "needs_layout_passes" in pltpu.CompilerParams.__dataclass_fields__:
        cp = dataclasses.replace(cp, needs_layout_passes=False)

    scratch = [pltpu.VMEM((CPP, K), jnp.int32)]
    scratch += [pltpu.VMEM((K,), jnp.int32) for _ in range(DEPTH)]
    scratch += [pltpu.VMEM((K,), jnp.int32) for _ in range(DEPTH)]
    scratch += [pltpu.VMEM((K, dh), jnp.float32) for _ in range(DEPTH)]
    scratch += [pltpu.VMEM((K, dh), jnp.float32) for _ in range(DEPTH)]
    scratch += [
        pltpu.VMEM((SPP, dh), jnp.float32),
        pltpu.VMEM((SPP, dh), jnp.float32),
        pltpu.VMEM_SHARED((NSUB * SPP, dh), jnp.float32),
        pltpu.VMEM_SHARED((NSUB * SPP, dh), jnp.float32),
    ]
    scratch += [pltpu.SemaphoreType.DMA] * (4 * DEPTH)

    @functools.partial(
        pl.kernel,
        out_type=jax.ShapeDtypeStruct((B, dh), jnp.float32),
        mesh=mesh,
        compiler_params=cp,
        scratch_types=scratch,
    )
    def k(idx_hbm, item_hbm, cat_hbm, i2c_hbm, out_hbm, idx_slab, *rest):
        cidx = rest[0:DEPTH]
        sess = rest[DEPTH:2 * DEPTH]
        ibuf = rest[2 * DEPTH:3 * DEPTH]
        cbuf = rest[3 * DEPTH:4 * DEPTH]
        icomb, ccomb, iacc, cacc = rest[4 * DEPTH:4 * DEPTH + 4]
        sems = rest[4 * DEPTH + 4:]
        semL = sems[0:DEPTH]
        semI = sems[DEPTH:2 * DEPTH]
        semC = sems[2 * DEPTH:3 * DEPTH]
        semS = sems[3 * DEPTH:4 * DEPTH]

        sid = lax.axis_index("s")
        wid = sid * 2 + lax.axis_index("c")

        zero = jnp.zeros((NLANE,), jnp.float32)
        iota = lax.iota(jnp.int32, NLANE)
        inv_l = jnp.float32(1.0 / L)
        alpha = jnp.float32(ALPHA)
        my_rows = pl.ds(sid * SPP, SPP)

        def gather_pair(r, j):
            idx_row = idx_slab.at[r]
            pltpu.async_copy(i2c_hbm.at[idx_row], cidx[j], semL[j])
            pltpu.async_copy(item_hbm.at[idx_row], ibuf[j], semI[j])

        def cat_stage(r, j, h):
            for g in range(K // NLANE):
                tok = iota + ((h * CPP + r) * K + g * NLANE)
                sess[j].at[pl.ds(g * NLANE, NLANE)][...] = (
                    sid * SPP + (tok // L - h * SPP))
            pltpu.make_async_copy(i2c_hbm.at[idx_slab.at[r]], cidx[j],
                                  semL[j]).wait()
            pltpu.async_copy(cat_hbm.at[cidx[j]], cbuf[j], semC[j])

        def scatter_stage(r, j):
            pltpu.make_async_copy(item_hbm.at[idx_slab.at[r]], ibuf[j],
                                  semI[j]).wait()
            pltpu.make_async_copy(cat_hbm.at[cidx[j]], cbuf[j], semC[j]).wait()
            pltpu.async_copy(ibuf[j], iacc.at[sess[j]], semS[j], add=True)
            pltpu.async_copy(cbuf[j], cacc.at[sess[j]], semS[j], add=True)

        def sa_drain(j):
            pltpu.make_async_copy(ibuf[j], iacc.at[sess[j]], semS[j]).wait()
            pltpu.make_async_copy(cbuf[j], cacc.at[sess[j]], semS[j]).wait()

        for h in range(PASSES):
            pltpu.sync_copy(idx_hbm.at[wid, pl.ds(h * CPP, CPP)], idx_slab)

            @pl.loop(0, SPP)
            def _(s):
                for d in range(dh // NLANE):
                    icomb.at[s, pl.ds(d * NLANE, NLANE)][...] = zero

            pltpu.sync_copy(icomb, iacc.at[my_rows])
            pltpu.sync_copy(icomb, cacc.at[my_rows])

            gather_pair(0, 0)
            gather_pair(1, 1)
            cat_stage(0, 0, h)

            @pl.loop(0, CPP // DEPTH)
            def _(t):
                for j in range(DEPTH):
                    r = DEPTH * t + j
                    sj2 = (j + 2) % DEPTH
                    sj1 = (j + 1) % DEPTH

                    @pl.when(r >= DEPTH - 2)
                    def _(sj2=sj2):
                        sa_drain(sj2)

                    @pl.when(r <= CPP - 3)
                    def _(r=r, sj2=sj2):
                        gather_pair(r + 2, sj2)

                    @pl.when(r <= CPP - 2)
                    def _(r=r, sj1=sj1):
                        cat_stage(r + 1, sj1, h)

                    scatter_stage(r, j)

            for tail in range(DEPTH - 2, 0, -1):
                sa_drain((CPP - tail) % DEPTH)

            pltpu.sync_copy(iacc.at[my_rows], icomb)
            pltpu.sync_copy(cacc.at[my_rows], ccomb)

            @pl.loop(0, SPP)
            def _(s):
                for d in range(dh // NLANE):
                    slc = (s, pl.ds(d * NLANE, NLANE))
                    a = icomb.at[slc][...]
                    b = ccomb.at[slc][...]
                    icomb.at[slc][...] = (a + alpha * b) * inv_l

            pltpu.sync_copy(icomb, out_hbm.at[pl.ds(wid * SPW + h * SPP, SPP)])

    return k


DH = D // 2
_k = _build(DH)


def kernel(indices, item_table, cat_table, item_to_cat):
    idx3 = indices.reshape(NW, NCHUNK, K)
    out_l = _k(idx3, item_table[:, :DH], cat_table[:, :DH], item_to_cat)
    out_r = _k(idx3, item_table[:, DH:], cat_table[:, DH:], item_to_cat)
    return jnp.concatenate([out_l, out_r], axis=1)


# speedup vs baseline: 1.9207x; 1.9207x over previous
import dataclasses
import functools

import jax
import jax.numpy as jnp
from jax import lax
from jax.experimental import pallas as pl
from jax.experimental.pallas import tpu as pltpu
from jax.experimental.pallas import tpu_sc as plsc

B = 16384
L = 50
D = 64
ALPHA = 0.1

NW = 32
NSUB = 16
TPW = B * L // NW
K = 128
NCHUNK = TPW // K
SPW = B // NW
PASSES = 4
SPP = SPW // PASSES
CPP = NCHUNK // PASSES
NLANE = 16
DEPTH = 5


def _params():
    cp = pltpu.CompilerParams(use_tc_tiling_on_sc=False)
    if "needs_layout_passes" in pltpu.CompilerParams.__dataclass_fields__:
        cp = dataclasses.replace(cp, needs_layout_passes=False)
    return cp


def _sess_stores(sess_ref, sid, r, h, iota):
    for g in range(K // NLANE):
        tok = iota + ((h * CPP + r) * K + g * NLANE)
        sess_ref.at[pl.ds(g * NLANE, NLANE)][...] = (
            sid * SPP + (tok // L - h * SPP))


def _build_cat():
    mesh = plsc.VectorSubcoreMesh(core_axis_name="c", subcore_axis_name="s")
    scratch = [pltpu.VMEM((CPP, K), jnp.int32)]
    scratch += [pltpu.VMEM((K,), jnp.int32) for _ in range(DEPTH)]
    scratch += [pltpu.VMEM((K,), jnp.int32) for _ in range(DEPTH)]
    scratch += [pltpu.VMEM((K, D), jnp.float32) for _ in range(DEPTH)]
    scratch += [
        pltpu.VMEM((SPP, D), jnp.float32),
        pltpu.VMEM_SHARED((NSUB * SPP, D), jnp.float32),
    ]
    scratch += [pltpu.SemaphoreType.DMA] * (3 * DEPTH)

    @functools.partial(
        pl.kernel,
        out_type=jax.ShapeDtypeStruct((B, D), jnp.float32),
        mesh=mesh,
        compiler_params=_params(),
        scratch_types=scratch,
    )
    def ka(idx_hbm, cat_hbm, i2c_hbm, out_hbm, idx_slab, *rest):
        cidx = rest[0:DEPTH]
        sess = rest[DEPTH:2 * DEPTH]
        cbuf = rest[2 * DEPTH:3 * DEPTH]
        comb, cacc = rest[3 * DEPTH:3 * DEPTH + 2]
        sems = rest[3 * DEPTH + 2:]
        semL = sems[0:DEPTH]
        semC = sems[DEPTH:2 * DEPTH]
        semS = sems[2 * DEPTH:3 * DEPTH]

        sid = lax.axis_index("s")
        wid = sid * 2 + lax.axis_index("c")
        zero = jnp.zeros((NLANE,), jnp.float32)
        iota = lax.iota(jnp.int32, NLANE)
        my_rows = pl.ds(sid * SPP, SPP)

        def g1(r, j):
            pltpu.async_copy(i2c_hbm.at[idx_slab.at[r]], cidx[j], semL[j])

        def g2(r, j, h):
            _sess_stores(sess[j], sid, r, h, iota)
            pltpu.make_async_copy(i2c_hbm.at[idx_slab.at[r]], cidx[j],
                                  semL[j]).wait()
            pltpu.async_copy(cat_hbm.at[cidx[j]], cbuf[j], semC[j])

        def g3(r, j):
            pltpu.make_async_copy(cat_hbm.at[cidx[j]], cbuf[j], semC[j]).wait()
            pltpu.async_copy(cbuf[j], cacc.at[sess[j]], semS[j], add=True)

        def drain(j):
            pltpu.make_async_copy(cbuf[j], cacc.at[sess[j]], semS[j]).wait()

        for h in range(PASSES):
            pltpu.sync_copy(idx_hbm.at[wid, pl.ds(h * CPP, CPP)], idx_slab)

            @pl.loop(0, SPP)
            def _(s):
                for d in range(D // NLANE):
                    comb.at[s, pl.ds(d * NLANE, NLANE)][...] = zero

            pltpu.sync_copy(comb, cacc.at[my_rows])

            g1(0, 0)
            g1(1, 1)
            g2(0, 0, h)

            @pl.loop(0, CPP // DEPTH)
            def _(t):
                for j in range(DEPTH):
                    r = DEPTH * t + j
                    sj2 = (j + 2) % DEPTH
                    sj1 = (j + 1) % DEPTH

                    @pl.when(r >= DEPTH - 2)
                    def _(sj2=sj2):
                        drain(sj2)

                    @pl.when(r <= CPP - 3)
                    def _(r=r, sj2=sj2):
                        g1(r + 2, sj2)

                    @pl.when(r <= CPP - 2)
                    def _(r=r, sj1=sj1):
                        g2(r + 1, sj1, h)

                    g3(r, j)

            for tail in range(DEPTH - 2, 0, -1):
                drain((CPP - tail) % DEPTH)

            pltpu.sync_copy(cacc.at[my_rows], comb)
            pltpu.sync_copy(comb, out_hbm.at[pl.ds(wid * SPW + h * SPP, SPP)])

    return ka


def _build_item():
    mesh = plsc.VectorSubcoreMesh(core_axis_name="c", subcore_axis_name="s")
    scratch = [pltpu.VMEM((CPP, K), jnp.int32)]
    scratch += [pltpu.VMEM((K,), jnp.int32) for _ in range(DEPTH)]
    scratch += [pltpu.VMEM((K, D), jnp.float32) for _ in range(DEPTH)]
    scratch += [
        pltpu.VMEM((SPP, D), jnp.float32),
        pltpu.VMEM((SPP, D), jnp.float32),
        pltpu.VMEM_SHARED((NSUB * SPP, D), jnp.float32),
    ]
    scratch += [pltpu.SemaphoreType.DMA] * (2 * DEPTH)

    @functools.partial(
        pl.kernel,
        out_type=jax.ShapeDtypeStruct((B, D), jnp.float32),
        mesh=mesh,
        compiler_params=_params(),
        scratch_types=scratch,
    )
    def kb(idx_hbm, item_hbm, csum_hbm, out_hbm, idx_slab, *rest):
        sess = rest[0:DEPTH]
        ibuf = rest[DEPTH:2 * DEPTH]
        icomb, ccomb, iacc = rest[2 * DEPTH:2 * DEPTH + 3]
        sems = rest[2 * DEPTH + 3:]
        semI = sems[0:DEPTH]
        semS = sems[DEPTH:2 * DEPTH]

        sid = lax.axis_index("s")
        wid = sid * 2 + lax.axis_index("c")
        zero = jnp.zeros((NLANE,), jnp.float32)
        iota = lax.iota(jnp.int32, NLANE)
        inv_l = jnp.float32(1.0 / L)
        alpha = jnp.float32(ALPHA)
        my_rows = pl.ds(sid * SPP, SPP)

        def g1(r, j, h):
            _sess_stores(sess[j], sid, r, h, iota)
            pltpu.async_copy(item_hbm.at[idx_slab.at[r]], ibuf[j], semI[j])

        def g3(r, j):
            pltpu.make_async_copy(item_hbm.at[idx_slab.at[r]], ibuf[j],
                                  semI[j]).wait()
            pltpu.async_copy(ibuf[j], iacc.at[sess[j]], semS[j], add=True)

        def drain(j):
            pltpu.make_async_copy(ibuf[j], iacc.at[sess[j]], semS[j]).wait()

        for h in range(PASSES):
            pltpu.sync_copy(idx_hbm.at[wid, pl.ds(h * CPP, CPP)], idx_slab)

            @pl.loop(0, SPP)
            def _(s):
                for d in range(D // NLANE):
                    icomb.at[s, pl.ds(d * NLANE, NLANE)][...] = zero

            pltpu.sync_copy(icomb, iacc.at[my_rows])

            g1(0, 0, h)
            g1(1, 1, h)

            @pl.loop(0, CPP // DEPTH)
            def _(t):
                for j in range(DEPTH):
                    r = DEPTH * t + j
                    sj2 = (j + 2) % DEPTH

                    @pl.when(r >= DEPTH - 2)
                    def _(sj2=sj2):
                        drain(sj2)

                    @pl.when(r <= CPP - 3)
                    def _(r=r, sj2=sj2, h=h):
                        g1(r + 2, sj2, h)

                    g3(r, j)

            for tail in range(DEPTH - 2, 0, -1):
                drain((CPP - tail) % DEPTH)

            out_rows = pl.ds(wid * SPW + h * SPP, SPP)
            pltpu.sync_copy(iacc.at[my_rows], icomb)
            pltpu.sync_copy(csum_hbm.at[out_rows], ccomb)

            @pl.loop(0, SPP)
            def _(s):
                for d in range(D // NLANE):
                    slc = (s, pl.ds(d * NLANE, NLANE))
                    a = icomb.at[slc][...]
                    b = ccomb.at[slc][...]
                    icomb.at[slc][...] = (a + alpha * b) * inv_l

            pltpu.sync_copy(icomb, out_hbm.at[out_rows])

    return kb


_ka = _build_cat()
_kb = _build_item()


def kernel(indices, item_table, cat_table, item_to_cat):
    idx3 = indices.reshape(NW, NCHUNK, K)
    csum = _ka(idx3, cat_table, item_to_cat)
    return _kb(idx3, item_table, csum)
